# baseline (speedup 1.0000x reference)
import jax
import jax.numpy as jnp
from jax import lax
from jax.experimental import pallas as pl
from jax.experimental.pallas import tpu as pltpu

_B, _N, _C = 4, 8192, 2048
_RT = 1024
_NT_CH = _N // (2 * _RT)


def _chunk_minarg(chunk, row0):
    lmin = jnp.min(chunk, axis=0)
    iota = lax.broadcasted_iota(jnp.int32, (_RT, _C), 0)
    masked = jnp.where(chunk == lmin[None, :], iota, _N)
    return lmin, jnp.min(masked, axis=0) + row0


def _tc_body(xa_ref, xb_ref, oidx_ref, val_ref, idx_ref):
    c = pl.program_id(1)
    amin, aarg = _chunk_minarg(xa_ref[0], (2 * c) * _RT)
    bmin, barg = _chunk_minarg(xb_ref[0], (2 * c + 1) * _RT)
    bb = bmin < amin
    lmin = jnp.where(bb, bmin, amin)
    larg = jnp.where(bb, barg, aarg)

    @pl.when(c == 0)
    def _():
        val_ref[0] = lmin
        idx_ref[0] = larg

    @pl.when(c > 0)
    def _():
        better = lmin < val_ref[0]
        val_ref[0] = jnp.where(better, lmin, val_ref[0])
        idx_ref[0] = jnp.where(better, larg, idx_ref[0])

    @pl.when(c == _NT_CH - 1)
    def _():
        oidx_ref[0, 0] = idx_ref[0]


def kernel(x):
    out = pl.pallas_call(
        _tc_body,
        grid=(_B, _NT_CH),
        in_specs=[
            pl.BlockSpec((1, _RT, _C), lambda b, c: (b, 2 * c, 0)),
            pl.BlockSpec((1, _RT, _C), lambda b, c: (b, 2 * c + 1, 0)),
        ],
        out_specs=pl.BlockSpec((1, 1, _C), lambda b, c: (b, 0, 0)),
        out_shape=jax.ShapeDtypeStruct((_B, 1, _C), jnp.int32),
        scratch_shapes=[
            pltpu.VMEM((1, _C), jnp.float32),
            pltpu.VMEM((1, _C), jnp.int32),
        ],
        compiler_params=pltpu.CompilerParams(
            dimension_semantics=("arbitrary", "arbitrary")
        ),
    )(x, x)
    return out.reshape(_B, _C).astype(jnp.int64)
